# Initial kernel scaffold; baseline (speedup 1.0000x reference)
import functools

import jax
import jax.numpy as jnp
from jax import lax
from jax.experimental import pallas as pl
from jax.experimental.pallas import tpu as pltpu
from jax.experimental.pallas import tpu_sc as plsc

N_NODES = 10000
N_EDGES = 320000
D = 128

NC = 2
NS = 16
NW = NC * NS
EPT = N_EDGES // NW
CH = 80
NCHUNK = EPT // CH
ROWS_PER_TILE = N_NODES // NS


def _sc_agg_body(x_hbm, src_hbm, dst_hbm, zeros_hbm, out_hbm,
                 src_all, dst_all, src_buf, dst_buf, rows, acc):
    c = lax.axis_index("c")
    s = lax.axis_index("s")
    wid = s * NC + c

    row0 = s * ROWS_PER_TILE
    pltpu.sync_copy(zeros_hbm.at[pl.ds(row0, ROWS_PER_TILE)],
                    acc.at[pl.ds(row0, ROWS_PER_TILE)])

    ebase = wid * EPT
    pltpu.sync_copy(src_hbm.at[pl.ds(ebase, EPT)], src_all)
    pltpu.sync_copy(dst_hbm.at[pl.ds(ebase, EPT)], dst_all)

    plsc.subcore_barrier()

    def chunk(j, carry):
        off = j * CH
        pltpu.sync_copy(src_all.at[pl.ds(off, CH)], src_buf)
        pltpu.sync_copy(x_hbm.at[src_buf], rows)
        pltpu.sync_copy(dst_all.at[pl.ds(off, CH)], dst_buf)
        pltpu.sync_copy(rows, acc.at[dst_buf], add=True)
        return carry

    lax.fori_loop(0, NCHUNK, chunk, 0)

    plsc.subcore_barrier()

    pltpu.sync_copy(acc.at[pl.ds(row0, ROWS_PER_TILE)],
                    out_hbm.at[c, pl.ds(row0, ROWS_PER_TILE)])


_sc_agg = pl.kernel(
    _sc_agg_body,
    out_type=jax.ShapeDtypeStruct((NC, N_NODES, D), jnp.float32),
    mesh=plsc.VectorSubcoreMesh(core_axis_name="c", subcore_axis_name="s"),
    scratch_types=[
        pltpu.VMEM((EPT,), jnp.int32),
        pltpu.VMEM((EPT,), jnp.int32),
        pltpu.VMEM((CH,), jnp.int32),
        pltpu.VMEM((CH,), jnp.int32),
        pltpu.VMEM((CH, D), jnp.float32),
        pltpu.VMEM_SHARED((N_NODES, D), jnp.float32),
    ],
)


def _mlp_body(x_ref, a_ref, w1_ref, b1_ref, w2_ref, b2_ref, o_ref):
    h = x_ref[...] + a_ref[0] + a_ref[1]
    t = jnp.dot(h, w1_ref[...], preferred_element_type=jnp.float32)
    t = jnp.maximum(t + b1_ref[...], 0.0)
    o = jnp.dot(t, w2_ref[...], preferred_element_type=jnp.float32)
    o_ref[...] = o + b2_ref[...]


BR = 1000


def _tc_mlp(x, agg, W1, b1, W2, b2):
    grid = (N_NODES // BR,)
    return pl.pallas_call(
        _mlp_body,
        grid=grid,
        in_specs=[
            pl.BlockSpec((BR, D), lambda i: (i, 0)),
            pl.BlockSpec((NC, BR, D), lambda i: (0, i, 0)),
            pl.BlockSpec((D, D), lambda i: (0, 0)),
            pl.BlockSpec((1, D), lambda i: (0, 0)),
            pl.BlockSpec((D, D), lambda i: (0, 0)),
            pl.BlockSpec((1, D), lambda i: (0, 0)),
        ],
        out_specs=pl.BlockSpec((BR, D), lambda i: (i, 0)),
        out_shape=jax.ShapeDtypeStruct((N_NODES, D), jnp.float32),
    )(x, agg, W1, b1.reshape(1, D), W2, b2.reshape(1, D))


def kernel(x, edge_index, W1a, b1a, W2a, b2a, W1b, b1b, W2b, b2b):
    src = edge_index[0].astype(jnp.int32)
    dst = edge_index[1].astype(jnp.int32)
    zeros = jnp.zeros((N_NODES, D), jnp.float32)

    agg = _sc_agg(x, src, dst, zeros)
    h1 = _tc_mlp(x, agg, W1a, b1a, W2a, b2a)
    agg2 = _sc_agg(h1, src, dst, zeros)
    return _tc_mlp(h1, agg2, W1b, b1b, W2b, b2b)


# speedup vs baseline: 5.1184x; 5.1184x over previous
import functools

import jax
import jax.numpy as jnp
from jax import lax
from jax.experimental import pallas as pl
from jax.experimental.pallas import tpu as pltpu
from jax.experimental.pallas import tpu_sc as plsc

N_NODES = 10000
N_EDGES = 320000
D = 128

NC = 2
NS = 16
NW = NC * NS
EPT = N_EDGES // NW
CH = 80
NCHUNK = EPT // CH
STRIPE = 624
STRIPE_LAST = 640


def _sc_agg_body(x_hbm, src_hbm, dst_hbm, zeros_hbm, out_hbm,
                 src_buf, dst_buf, rows, acc):
    c = lax.axis_index("c")
    s = lax.axis_index("s")
    wid = s * NC + c

    row0 = pl.multiple_of(s * STRIPE, 8)

    @pl.when(s < NS - 1)
    def _zero_main():
        pltpu.sync_copy(zeros_hbm.at[pl.ds(row0, STRIPE)],
                        acc.at[pl.ds(row0, STRIPE)])

    @pl.when(s == NS - 1)
    def _zero_last():
        pltpu.sync_copy(zeros_hbm.at[pl.ds((NS - 1) * STRIPE, STRIPE_LAST)],
                        acc.at[pl.ds((NS - 1) * STRIPE, STRIPE_LAST)])

    ebase = pl.multiple_of(wid * EPT, 8)

    plsc.subcore_barrier()

    def chunk(j, carry):
        off = pl.multiple_of(ebase + j * CH, 8)
        pltpu.sync_copy(src_hbm.at[pl.ds(off, CH)], src_buf)
        pltpu.sync_copy(x_hbm.at[src_buf], rows)
        pltpu.sync_copy(dst_hbm.at[pl.ds(off, CH)], dst_buf)
        pltpu.sync_copy(rows, acc.at[dst_buf], add=True)
        return carry

    lax.fori_loop(0, NCHUNK, chunk, 0)

    plsc.subcore_barrier()

    @pl.when(s < NS - 1)
    def _pub_main():
        pltpu.sync_copy(acc.at[pl.ds(row0, STRIPE)],
                        out_hbm.at[c, pl.ds(row0, STRIPE)])

    @pl.when(s == NS - 1)
    def _pub_last():
        pltpu.sync_copy(acc.at[pl.ds((NS - 1) * STRIPE, STRIPE_LAST)],
                        out_hbm.at[c, pl.ds((NS - 1) * STRIPE, STRIPE_LAST)])


_sc_agg = pl.kernel(
    _sc_agg_body,
    out_type=jax.ShapeDtypeStruct((NC, N_NODES, D), jnp.float32),
    mesh=plsc.VectorSubcoreMesh(core_axis_name="c", subcore_axis_name="s"),
    scratch_types=[
        pltpu.VMEM((CH,), jnp.int32),
        pltpu.VMEM((CH,), jnp.int32),
        pltpu.VMEM((CH, D), jnp.float32),
        pltpu.VMEM_SHARED((N_NODES, D), jnp.float32),
    ],
)


def _mlp_body(x_ref, a_ref, w1_ref, b1_ref, w2_ref, b2_ref, o_ref):
    h = x_ref[...] + a_ref[0] + a_ref[1]
    t = jnp.dot(h, w1_ref[...], preferred_element_type=jnp.float32)
    t = jnp.maximum(t + b1_ref[...], 0.0)
    o = jnp.dot(t, w2_ref[...], preferred_element_type=jnp.float32)
    o_ref[...] = o + b2_ref[...]


BR = 1000


def _tc_mlp(x, agg, W1, b1, W2, b2):
    grid = (N_NODES // BR,)
    return pl.pallas_call(
        _mlp_body,
        grid=grid,
        in_specs=[
            pl.BlockSpec((BR, D), lambda i: (i, 0)),
            pl.BlockSpec((NC, BR, D), lambda i: (0, i, 0)),
            pl.BlockSpec((D, D), lambda i: (0, 0)),
            pl.BlockSpec((1, D), lambda i: (0, 0)),
            pl.BlockSpec((D, D), lambda i: (0, 0)),
            pl.BlockSpec((1, D), lambda i: (0, 0)),
        ],
        out_specs=pl.BlockSpec((BR, D), lambda i: (i, 0)),
        out_shape=jax.ShapeDtypeStruct((N_NODES, D), jnp.float32),
    )(x, agg, W1, b1.reshape(1, D), W2, b2.reshape(1, D))


def kernel(x, edge_index, W1a, b1a, W2a, b2a, W1b, b1b, W2b, b2b):
    src = edge_index[0].astype(jnp.int32)
    dst = edge_index[1].astype(jnp.int32)
    zeros = jnp.zeros((N_NODES, D), jnp.float32)

    agg = _sc_agg(x, src, dst, zeros)
    h1 = _tc_mlp(x, agg, W1a, b1a, W2a, b2a)
    agg2 = _sc_agg(h1, src, dst, zeros)
    return _tc_mlp(h1, agg2, W1b, b1b, W2b, b2b)


# speedup vs baseline: 10.9649x; 2.1423x over previous
import functools

import jax
import jax.numpy as jnp
from jax import lax
from jax.experimental import pallas as pl
from jax.experimental.pallas import tpu as pltpu
from jax.experimental.pallas import tpu_sc as plsc

N_NODES = 10000
N_EDGES = 320000
D = 128

NC = 2
NS = 16
NW = NC * NS
EPT = N_EDGES // NW
CH = 128
NFULL = EPT // CH
TAIL = EPT - NFULL * CH
STRIPE = 624
STRIPE_LAST = 640


def _sc_agg_body(x_hbm, src_hbm, dst_hbm, zeros_hbm, out_hbm,
                 src0, src1, dst0, dst1, rows0, rows1,
                 src_t, dst_t, rows_t, acc,
                 sem_i0, sem_i1, sem_g0, sem_g1):
    c = lax.axis_index("c")
    s = lax.axis_index("s")
    wid = s * NC + c
    ebase = pl.multiple_of(wid * EPT, 8)

    src_b = (src0, src1)
    dst_b = (dst0, dst1)
    rows_b = (rows0, rows1)
    sem_i = (sem_i0, sem_i1)
    sem_g = (sem_g0, sem_g1)

    def start_idx(j, b):
        off = pl.multiple_of(ebase + j * CH, 8)
        pltpu.async_copy(src_hbm.at[pl.ds(off, CH)], src_b[b], sem_i[b])
        pltpu.async_copy(dst_hbm.at[pl.ds(off, CH)], dst_b[b], sem_i[b])

    def wait_idx(b):
        pltpu.make_async_copy(src_hbm.at[pl.ds(0, CH)], src_b[b], sem_i[b]).wait()
        pltpu.make_async_copy(dst_hbm.at[pl.ds(0, CH)], dst_b[b], sem_i[b]).wait()

    def start_gather(b):
        pltpu.async_copy(x_hbm.at[src_b[b]], rows_b[b], sem_g[b])

    def wait_gather(b):
        pltpu.make_async_copy(x_hbm.at[src_b[b]], rows_b[b], sem_g[b]).wait()

    start_idx(0, 0)
    start_idx(1, 1)

    row0 = pl.multiple_of(s * STRIPE, 8)

    @pl.when(s < NS - 1)
    def _zero_main():
        pltpu.sync_copy(zeros_hbm.at[pl.ds(row0, STRIPE)],
                        acc.at[pl.ds(row0, STRIPE)])

    @pl.when(s == NS - 1)
    def _zero_last():
        pltpu.sync_copy(zeros_hbm.at[pl.ds((NS - 1) * STRIPE, STRIPE_LAST)],
                        acc.at[pl.ds((NS - 1) * STRIPE, STRIPE_LAST)])

    plsc.subcore_barrier()

    wait_idx(0)
    start_gather(0)

    @pl.loop(0, NFULL, step=2)
    def _pair(p):
        for b in (0, 1):
            j = p + b
            b1 = 1 - b
            wait_gather(b)

            @pl.when(j + 1 < NFULL)
            def _next_gather():
                wait_idx(b1)
                start_gather(b1)

            pltpu.sync_copy(rows_b[b], acc.at[dst_b[b]], add=True)

            @pl.when(j + 2 < NFULL)
            def _prefetch_idx():
                start_idx(j + 2, b)

    toff = pl.multiple_of(ebase + NFULL * CH, 8)
    pltpu.sync_copy(src_hbm.at[pl.ds(toff, TAIL)], src_t)
    pltpu.sync_copy(dst_hbm.at[pl.ds(toff, TAIL)], dst_t)
    pltpu.sync_copy(x_hbm.at[src_t], rows_t)
    pltpu.sync_copy(rows_t, acc.at[dst_t], add=True)

    plsc.subcore_barrier()

    @pl.when(s < NS - 1)
    def _pub_main():
        pltpu.sync_copy(acc.at[pl.ds(row0, STRIPE)],
                        out_hbm.at[c, pl.ds(row0, STRIPE)])

    @pl.when(s == NS - 1)
    def _pub_last():
        pltpu.sync_copy(acc.at[pl.ds((NS - 1) * STRIPE, STRIPE_LAST)],
                        out_hbm.at[c, pl.ds((NS - 1) * STRIPE, STRIPE_LAST)])


_sc_agg = pl.kernel(
    _sc_agg_body,
    out_type=jax.ShapeDtypeStruct((NC, N_NODES, D), jnp.float32),
    mesh=plsc.VectorSubcoreMesh(core_axis_name="c", subcore_axis_name="s"),
    scratch_types=[
        pltpu.VMEM((CH,), jnp.int32),
        pltpu.VMEM((CH,), jnp.int32),
        pltpu.VMEM((CH,), jnp.int32),
        pltpu.VMEM((CH,), jnp.int32),
        pltpu.VMEM((CH, D), jnp.float32),
        pltpu.VMEM((CH, D), jnp.float32),
        pltpu.VMEM((TAIL,), jnp.int32),
        pltpu.VMEM((TAIL,), jnp.int32),
        pltpu.VMEM((TAIL, D), jnp.float32),
        pltpu.VMEM_SHARED((N_NODES, D), jnp.float32),
        pltpu.SemaphoreType.DMA,
        pltpu.SemaphoreType.DMA,
        pltpu.SemaphoreType.DMA,
        pltpu.SemaphoreType.DMA,
    ],
)


def _mlp_body(x_ref, a_ref, w1_ref, b1_ref, w2_ref, b2_ref, o_ref):
    h = x_ref[...] + a_ref[0] + a_ref[1]
    t = jnp.dot(h, w1_ref[...], preferred_element_type=jnp.float32)
    t = jnp.maximum(t + b1_ref[...], 0.0)
    o = jnp.dot(t, w2_ref[...], preferred_element_type=jnp.float32)
    o_ref[...] = o + b2_ref[...]


BR = 1000


def _tc_mlp(x, agg, W1, b1, W2, b2):
    grid = (N_NODES // BR,)
    return pl.pallas_call(
        _mlp_body,
        grid=grid,
        in_specs=[
            pl.BlockSpec((BR, D), lambda i: (i, 0)),
            pl.BlockSpec((NC, BR, D), lambda i: (0, i, 0)),
            pl.BlockSpec((D, D), lambda i: (0, 0)),
            pl.BlockSpec((1, D), lambda i: (0, 0)),
            pl.BlockSpec((D, D), lambda i: (0, 0)),
            pl.BlockSpec((1, D), lambda i: (0, 0)),
        ],
        out_specs=pl.BlockSpec((BR, D), lambda i: (i, 0)),
        out_shape=jax.ShapeDtypeStruct((N_NODES, D), jnp.float32),
    )(x, agg, W1, b1.reshape(1, D), W2, b2.reshape(1, D))


def kernel(x, edge_index, W1a, b1a, W2a, b2a, W1b, b1b, W2b, b2b):
    src = edge_index[0].astype(jnp.int32)
    dst = edge_index[1].astype(jnp.int32)
    zeros = jnp.zeros((N_NODES, D), jnp.float32)

    agg = _sc_agg(x, src, dst, zeros)
    h1 = _tc_mlp(x, agg, W1a, b1a, W2a, b2a)
    agg2 = _sc_agg(h1, src, dst, zeros)
    return _tc_mlp(h1, agg2, W1b, b1b, W2b, b2b)
